# baseline (speedup 1.0000x reference)
import functools

import jax
import jax.numpy as jnp
import numpy as np
from jax import lax
from jax.experimental import pallas as pl
from jax.experimental.pallas import tpu as pltpu
from jax.experimental.pallas import tpu_sc as plsc

_BATCH = 16384
_EMB = 32
_HID = 64
_NC = 2
_NS = 16
_NW = _NC * _NS
_BPW = _BATCH // _NW
_CHUNK = 128
_NK = _BPW // _CHUNK
_NP = _EMB // 2

_ROWS = 1000000
_RPAD = 1 << 20
_DC = 1 << 16
_NJ = _RPAD // _DC
_NJIN = -(-_ROWS // _DC)

_MASKHI = -65536


def _pack_body(u_ref, i_ref, *o_refs):
    for pp in range(4):
        for t, src in enumerate((u_ref, i_ref)):
            a = lax.bitcast_convert_type(src[2 * pp, :], jnp.int32)
            b = lax.bitcast_convert_type(src[2 * pp + 1, :], jnp.int32)
            o_refs[4 * t + pp][...] = jnp.bitwise_or(
                jnp.bitwise_and(a, _MASKHI),
                lax.shift_right_logical(b, 16))


_pack = pl.pallas_call(
    _pack_body,
    grid=(4, _NJIN),
    in_specs=[
        pl.BlockSpec((8, _DC), lambda db, j: (db, j)),
        pl.BlockSpec((8, _DC), lambda db, j: (db, j)),
    ],
    out_specs=[pl.BlockSpec((_DC,), lambda db, j: (db * _NJ + j,))
               for _ in range(8)],
    out_shape=[jax.ShapeDtypeStruct((4 * _RPAD,), jnp.int32)
               for _ in range(8)],
)


def _sc_gather_body(*refs):
    (uidx, iidx, u_out, i_out,
     uidx_v, iidx_v, urows_v, irows_v, sem) = refs[8:]
    ustag = refs[0:4]
    istag = refs[4:8]
    wid = lax.axis_index("s") * _NC + lax.axis_index("c")
    base = wid * _BPW
    row0 = wid * _NK
    pltpu.sync_copy(uidx.at[pl.ds(row0, _NK)], uidx_v)
    pltpu.sync_copy(iidx.at[pl.ds(row0, _NK)], iidx_v)

    def fire(p):
        db, pp = p // 4, p % 4
        cs = []
        for k in range(_NK):
            cs.append(pltpu.async_copy(
                ustag[pp].at[db].at[uidx_v.at[k]],
                urows_v.at[p, pl.ds(k * _CHUNK, _CHUNK)], sem))
            cs.append(pltpu.async_copy(
                istag[pp].at[db].at[iidx_v.at[k]],
                irows_v.at[p, pl.ds(k * _CHUNK, _CHUNK)], sem))
        return cs

    depth = 4
    pend = [fire(p) for p in range(depth)]
    for p in range(depth, _NP):
        nxt = fire(p)
        for c in pend.pop(0):
            c.wait()
        pend.append(nxt)
    for grp in pend:
        for c in grp:
            c.wait()

    pltpu.sync_copy(urows_v, u_out.at[:, pl.ds(base, _BPW)])
    pltpu.sync_copy(irows_v, i_out.at[:, pl.ds(base, _BPW)])


_sc_gather = functools.partial(
    pl.kernel,
    mesh=plsc.VectorSubcoreMesh(core_axis_name="c", subcore_axis_name="s"),
    out_type=[
        jax.ShapeDtypeStruct((_NP, _BATCH), jnp.int32),
        jax.ShapeDtypeStruct((_NP, _BATCH), jnp.int32),
    ],
    scratch_types=[
        pltpu.VMEM((_NK, _CHUNK), jnp.int32),
        pltpu.VMEM((_NK, _CHUNK), jnp.int32),
        pltpu.VMEM((_NP, _BPW), jnp.int32),
        pltpu.VMEM((_NP, _BPW), jnp.int32),
        pltpu.SemaphoreType.DMA,
    ],
    compiler_params=pltpu.CompilerParams(use_tc_tiling_on_sc=False),
)(_sc_gather_body)


_BN = 2048


def _unpack(x):
    hi = lax.bitcast_convert_type(jnp.bitwise_and(x, _MASKHI), jnp.float32)
    lo = lax.bitcast_convert_type(lax.shift_left(x, 16), jnp.float32)
    return jnp.concatenate([hi, lo], axis=0)


def _mlp_body(w1u_ref, w1i_ref, b1_ref, w2_ref, b2_ref, u_ref, i_ref, o_ref):
    u = _unpack(u_ref[...])
    i = _unpack(i_ref[...])
    h = jnp.dot(w1u_ref[...], u, preferred_element_type=jnp.float32)
    h = h + jnp.dot(w1i_ref[...], i, preferred_element_type=jnp.float32)
    h = jnp.maximum(h + b1_ref[...], 0.0)
    o_ref[...] = jnp.dot(w2_ref[...], h, preferred_element_type=jnp.float32) + b2_ref[...]


_mlp = pl.pallas_call(
    _mlp_body,
    grid=(_BATCH // _BN,),
    in_specs=[
        pl.BlockSpec((_HID, _EMB), lambda n: (0, 0)),
        pl.BlockSpec((_HID, _EMB), lambda n: (0, 0)),
        pl.BlockSpec((_HID, 1), lambda n: (0, 0)),
        pl.BlockSpec((1, _HID), lambda n: (0, 0)),
        pl.BlockSpec((1, 1), lambda n: (0, 0)),
        pl.BlockSpec((_NP, _BN), lambda n: (0, n)),
        pl.BlockSpec((_NP, _BN), lambda n: (0, n)),
    ],
    out_specs=pl.BlockSpec((1, _BN), lambda n: (0, n)),
    out_shape=jax.ShapeDtypeStruct((1, _BATCH), jnp.float32),
)

_PERM = ([8 * (p // 4) + 2 * (p % 4) for p in range(_NP)]
         + [8 * (p // 4) + 2 * (p % 4) + 1 for p in range(_NP)])


def kernel(users, items, user_table, item_table, W1, b1, W2, b2):
    uidx = users.reshape(_NW * _NK, _CHUNK)
    iidx = items.reshape(_NW * _NK, _CHUNK)
    stags = _pack(user_table.T, item_table.T)
    stags = [s.reshape(4, _RPAD) for s in stags]
    u_t, i_t = _sc_gather(*stags, uidx, iidx)
    perm = np.array(_PERM)
    w1u = W1[:, :_EMB][:, perm]
    w1i = W1[:, _EMB:][:, perm]
    out = _mlp(w1u, w1i, b1.reshape(_HID, 1), W2.reshape(1, _HID),
               b2.reshape(1, 1), u_t, i_t)
    return out.reshape(_BATCH)


# speedup vs baseline: 1.2205x; 1.2205x over previous
import functools

import jax
import jax.numpy as jnp
from jax import lax
from jax.experimental import pallas as pl
from jax.experimental.pallas import tpu as pltpu
from jax.experimental.pallas import tpu_sc as plsc

_BATCH = 16384
_EMB = 32
_HID = 64
_NC = 2
_NS = 16
_NW = _NC * _NS
_BPW = _BATCH // _NW
_CHUNK = 128
_NK = _BPW // _CHUNK

_ROWS = 1000000
_RPAD = 1 << 20
_DC = 1 << 17
_NJ = _RPAD // _DC
_NJIN = -(-_ROWS // _DC)


def _detile_body(u_ref, i_ref, *o_refs):
    for dd in range(8):
        o_refs[dd][...] = u_ref[dd, :]
        o_refs[8 + dd][...] = i_ref[dd, :]


_detile = pl.pallas_call(
    _detile_body,
    grid=(4, _NJIN),
    in_specs=[
        pl.BlockSpec((8, _DC), lambda db, j: (db, j)),
        pl.BlockSpec((8, _DC), lambda db, j: (db, j)),
    ],
    out_specs=[pl.BlockSpec((_DC,), lambda db, j: (db * _NJ + j,))
               for _ in range(16)],
    out_shape=[jax.ShapeDtypeStruct((4 * _RPAD,), jnp.float32)
               for _ in range(16)],
)


def _sc_gather_body(*refs):
    (uidx, iidx, u_out, i_out,
     uidx_v, iidx_v, urows_v, irows_v, sem) = refs[16:]
    ustag = refs[0:8]
    istag = refs[8:16]
    wid = lax.axis_index("s") * _NC + lax.axis_index("c")
    base = wid * _BPW
    row0 = wid * _NK
    pltpu.sync_copy(uidx.at[pl.ds(row0, _NK)], uidx_v)
    pltpu.sync_copy(iidx.at[pl.ds(row0, _NK)], iidx_v)

    def fire(d):
        db, dd = d // 8, d % 8
        cs = []
        for k in range(_NK):
            cs.append(pltpu.async_copy(
                ustag[dd].at[db].at[uidx_v.at[k]],
                urows_v.at[d, pl.ds(k * _CHUNK, _CHUNK)], sem))
            cs.append(pltpu.async_copy(
                istag[dd].at[db].at[iidx_v.at[k]],
                irows_v.at[d, pl.ds(k * _CHUNK, _CHUNK)], sem))
        return cs

    depth = 4
    pend = [fire(d) for d in range(depth)]
    for d in range(depth, _EMB):
        nxt = fire(d)
        for c in pend.pop(0):
            c.wait()
        pend.append(nxt)
    for grp in pend:
        for c in grp:
            c.wait()

    pltpu.sync_copy(urows_v, u_out.at[:, pl.ds(base, _BPW)])
    pltpu.sync_copy(irows_v, i_out.at[:, pl.ds(base, _BPW)])


_sc_gather = functools.partial(
    pl.kernel,
    mesh=plsc.VectorSubcoreMesh(core_axis_name="c", subcore_axis_name="s"),
    out_type=[
        jax.ShapeDtypeStruct((_EMB, _BATCH), jnp.float32),
        jax.ShapeDtypeStruct((_EMB, _BATCH), jnp.float32),
    ],
    scratch_types=[
        pltpu.VMEM((_NK, _CHUNK), jnp.int32),
        pltpu.VMEM((_NK, _CHUNK), jnp.int32),
        pltpu.VMEM((_EMB, _BPW), jnp.float32),
        pltpu.VMEM((_EMB, _BPW), jnp.float32),
        pltpu.SemaphoreType.DMA,
    ],
    compiler_params=pltpu.CompilerParams(use_tc_tiling_on_sc=False),
)(_sc_gather_body)


_BN = 2048


def _mlp_body(w1u_ref, w1i_ref, b1_ref, w2_ref, b2_ref, u_ref, i_ref, o_ref):
    h = jnp.dot(w1u_ref[...], u_ref[...], preferred_element_type=jnp.float32)
    h = h + jnp.dot(w1i_ref[...], i_ref[...], preferred_element_type=jnp.float32)
    h = jnp.maximum(h + b1_ref[...], 0.0)
    o_ref[...] = jnp.dot(w2_ref[...], h, preferred_element_type=jnp.float32) + b2_ref[...]


_mlp = pl.pallas_call(
    _mlp_body,
    grid=(_BATCH // _BN,),
    in_specs=[
        pl.BlockSpec((_HID, _EMB), lambda n: (0, 0)),
        pl.BlockSpec((_HID, _EMB), lambda n: (0, 0)),
        pl.BlockSpec((_HID, 1), lambda n: (0, 0)),
        pl.BlockSpec((1, _HID), lambda n: (0, 0)),
        pl.BlockSpec((1, 1), lambda n: (0, 0)),
        pl.BlockSpec((_EMB, _BN), lambda n: (0, n)),
        pl.BlockSpec((_EMB, _BN), lambda n: (0, n)),
    ],
    out_specs=pl.BlockSpec((1, _BN), lambda n: (0, n)),
    out_shape=jax.ShapeDtypeStruct((1, _BATCH), jnp.float32),
)


def kernel(users, items, user_table, item_table, W1, b1, W2, b2):
    uidx = users.reshape(_NW * _NK, _CHUNK)
    iidx = items.reshape(_NW * _NK, _CHUNK)
    stags = _detile(user_table.T, item_table.T)
    stags = [s.reshape(4, _RPAD) for s in stags]
    u_t, i_t = _sc_gather(*stags, uidx, iidx)
    w1u = W1[:, :_EMB]
    w1i = W1[:, _EMB:]
    out = _mlp(w1u, w1i, b1.reshape(_HID, 1), W2.reshape(1, _HID),
               b2.reshape(1, 1), u_t, i_t)
    return out.reshape(_BATCH)


# speedup vs baseline: 1.2211x; 1.0005x over previous
import functools

import jax
import jax.numpy as jnp
from jax import lax
from jax.experimental import pallas as pl
from jax.experimental.pallas import tpu as pltpu
from jax.experimental.pallas import tpu_sc as plsc

_BATCH = 16384
_EMB = 32
_HID = 64
_NC = 2
_NS = 16
_NW = _NC * _NS
_BPW = _BATCH // _NW
_CHUNK = 128
_NK = _BPW // _CHUNK

_ROWS = 1000000
_RPAD = 1 << 20
_DC = 1 << 17
_NJ = _RPAD // _DC
_NJIN = -(-_ROWS // _DC)


def _detile_body(u_ref, i_ref, *o_refs):
    for dd in range(8):
        o_refs[dd][...] = u_ref[dd, :]
        o_refs[8 + dd][...] = i_ref[dd, :]


_detile = pl.pallas_call(
    _detile_body,
    grid=(4, _NJIN),
    in_specs=[
        pl.BlockSpec((8, _DC), lambda db, j: (db, j)),
        pl.BlockSpec((8, _DC), lambda db, j: (db, j)),
    ],
    out_specs=[pl.BlockSpec((_DC,), lambda db, j: (db * _NJ + j,))
               for _ in range(16)],
    out_shape=[jax.ShapeDtypeStruct((4 * _RPAD,), jnp.float32)
               for _ in range(16)],
)


def _sc_gather_body(*refs):
    (uidx, iidx, u_out, i_out,
     uidx_v, iidx_v, urows_v, irows_v, sem) = refs[16:]
    ustag = refs[0:8]
    istag = refs[8:16]
    wid = lax.axis_index("s") * _NC + lax.axis_index("c")
    base = wid * _BPW
    row0 = wid * _NK
    pltpu.sync_copy(uidx.at[pl.ds(row0, _NK)], uidx_v)
    pltpu.sync_copy(iidx.at[pl.ds(row0, _NK)], iidx_v)

    def fire(d):
        db, dd = d // 8, d % 8
        cs = []
        for k in range(_NK):
            cs.append(pltpu.async_copy(
                ustag[dd].at[db].at[uidx_v.at[k]],
                urows_v.at[d, pl.ds(k * _CHUNK, _CHUNK)], sem))
            cs.append(pltpu.async_copy(
                istag[dd].at[db].at[iidx_v.at[k]],
                irows_v.at[d, pl.ds(k * _CHUNK, _CHUNK)], sem))
        return cs

    depth = 4
    pend = [fire(d) for d in range(depth)]
    for d in range(depth, _EMB):
        nxt = fire(d)
        for c in pend.pop(0):
            c.wait()
        pend.append(nxt)
    for grp in pend:
        for c in grp:
            c.wait()

    pltpu.sync_copy(urows_v, u_out.at[:, pl.ds(base, _BPW)])
    pltpu.sync_copy(irows_v, i_out.at[:, pl.ds(base, _BPW)])


_sc_gather = functools.partial(
    pl.kernel,
    mesh=plsc.VectorSubcoreMesh(core_axis_name="c", subcore_axis_name="s"),
    out_type=[
        jax.ShapeDtypeStruct((_EMB, _BATCH), jnp.float32),
        jax.ShapeDtypeStruct((_EMB, _BATCH), jnp.float32),
    ],
    scratch_types=[
        pltpu.VMEM((_NK, _CHUNK), jnp.int32),
        pltpu.VMEM((_NK, _CHUNK), jnp.int32),
        pltpu.VMEM((_EMB, _BPW), jnp.float32),
        pltpu.VMEM((_EMB, _BPW), jnp.float32),
        pltpu.SemaphoreType.DMA,
    ],
    compiler_params=pltpu.CompilerParams(use_tc_tiling_on_sc=False),
)(_sc_gather_body)


_BN = 2048


def _mlp_body(w1u_ref, w1i_ref, b1_ref, w2_ref, b2_ref, u_ref, i_ref, o_ref):
    u = u_ref[...]
    i = i_ref[...]
    h = jnp.dot(w1u_ref[...], u, preferred_element_type=jnp.float32)
    h = h + jnp.dot(w1i_ref[...], i, preferred_element_type=jnp.float32)
    h = jnp.maximum(h + b1_ref[...], 0.0)
    o_ref[...] = jnp.dot(w2_ref[...], h, preferred_element_type=jnp.float32) + b2_ref[...]


_mlp = pl.pallas_call(
    _mlp_body,
    grid=(_BATCH // _BN,),
    in_specs=[
        pl.BlockSpec((_HID, _EMB), lambda n: (0, 0)),
        pl.BlockSpec((_HID, _EMB), lambda n: (0, 0)),
        pl.BlockSpec((_HID, 1), lambda n: (0, 0)),
        pl.BlockSpec((1, _HID), lambda n: (0, 0)),
        pl.BlockSpec((1, 1), lambda n: (0, 0)),
        pl.BlockSpec((_EMB, _BN), lambda n: (0, n)),
        pl.BlockSpec((_EMB, _BN), lambda n: (0, n)),
    ],
    out_specs=pl.BlockSpec((1, _BN), lambda n: (0, n)),
    out_shape=jax.ShapeDtypeStruct((1, _BATCH), jnp.float32),
)


def kernel(users, items, user_table, item_table, W1, b1, W2, b2):
    uidx = users.reshape(_NW * _NK, _CHUNK)
    iidx = items.reshape(_NW * _NK, _CHUNK)
    stags = _detile(user_table.T, item_table.T)
    stags = [s.reshape(4, _RPAD) for s in stags]
    u_t, i_t = _sc_gather(*stags, uidx, iidx)
    w1u = W1[:, :_EMB]
    w1i = W1[:, _EMB:]
    out = _mlp(w1u, w1i, b1.reshape(_HID, 1), W2.reshape(1, _HID),
               b2.reshape(1, 1), u_t, i_t)
    return out.reshape(_BATCH)


# speedup vs baseline: 1.2746x; 1.0438x over previous
import functools

import jax
import jax.numpy as jnp
from jax import lax
from jax.experimental import pallas as pl
from jax.experimental.pallas import tpu as pltpu
from jax.experimental.pallas import tpu_sc as plsc

_BATCH = 16384
_EMB = 32
_HID = 64
_NC = 2
_NS = 16
_NW = _NC * _NS
_BPW = _BATCH // _NW
_CHUNK = 128
_NK = _BPW // _CHUNK

_ROWS = 1000000
_RPAD = 1 << 20
_DC = 1 << 17
_NJ = _RPAD // _DC
_NJIN = -(-_ROWS // _DC)


def _detile_body(t_ref, *o_refs):
    for dd in range(8):
        o_refs[dd][...] = t_ref[dd, :]


_detile = pl.pallas_call(
    _detile_body,
    grid=(4, _NJIN),
    in_specs=[pl.BlockSpec((8, _DC), lambda db, j: (db, j))],
    out_specs=[pl.BlockSpec((_DC,), lambda db, j: (db * _NJ + j,))
               for _ in range(8)],
    out_shape=[jax.ShapeDtypeStruct((4 * _RPAD,), jnp.float32)
               for _ in range(8)],
)


def _sc_gather_body(*refs):
    (idx, t_out, idx_v, rows_v, sem) = refs[8:]
    stag = refs[0:8]
    wid = lax.axis_index("s") * _NC + lax.axis_index("c")
    base = wid * _BPW
    row0 = wid * _NK
    pltpu.sync_copy(idx.at[pl.ds(row0, _NK)], idx_v)

    def fire(d):
        db, dd = d // 8, d % 8
        cs = []
        for k in range(_NK):
            cs.append(pltpu.async_copy(
                stag[dd].at[db].at[idx_v.at[k]],
                rows_v.at[d, pl.ds(k * _CHUNK, _CHUNK)], sem))
        return cs

    depth = 8
    pend = [fire(d) for d in range(depth)]
    for d in range(depth, _EMB):
        nxt = fire(d)
        for c in pend.pop(0):
            c.wait()
        pend.append(nxt)
    for grp in pend:
        for c in grp:
            c.wait()

    pltpu.sync_copy(rows_v, t_out.at[:, pl.ds(base, _BPW)])


_sc_gather = functools.partial(
    pl.kernel,
    mesh=plsc.VectorSubcoreMesh(core_axis_name="c", subcore_axis_name="s"),
    out_type=jax.ShapeDtypeStruct((_EMB, _BATCH), jnp.float32),
    scratch_types=[
        pltpu.VMEM((_NK, _CHUNK), jnp.int32),
        pltpu.VMEM((_EMB, _BPW), jnp.float32),
        pltpu.SemaphoreType.DMA,
    ],
    compiler_params=pltpu.CompilerParams(use_tc_tiling_on_sc=False),
)(_sc_gather_body)


_BN = 2048


def _mlp_body(w1u_ref, w1i_ref, b1_ref, w2_ref, b2_ref, u_ref, i_ref, o_ref):
    u = u_ref[...]
    i = i_ref[...]
    h = jnp.dot(w1u_ref[...], u, preferred_element_type=jnp.float32)
    h = h + jnp.dot(w1i_ref[...], i, preferred_element_type=jnp.float32)
    h = jnp.maximum(h + b1_ref[...], 0.0)
    o_ref[...] = jnp.dot(w2_ref[...], h, preferred_element_type=jnp.float32) + b2_ref[...]


_mlp = pl.pallas_call(
    _mlp_body,
    grid=(_BATCH // _BN,),
    in_specs=[
        pl.BlockSpec((_HID, _EMB), lambda n: (0, 0)),
        pl.BlockSpec((_HID, _EMB), lambda n: (0, 0)),
        pl.BlockSpec((_HID, 1), lambda n: (0, 0)),
        pl.BlockSpec((1, _HID), lambda n: (0, 0)),
        pl.BlockSpec((1, 1), lambda n: (0, 0)),
        pl.BlockSpec((_EMB, _BN), lambda n: (0, n)),
        pl.BlockSpec((_EMB, _BN), lambda n: (0, n)),
    ],
    out_specs=pl.BlockSpec((1, _BN), lambda n: (0, n)),
    out_shape=jax.ShapeDtypeStruct((1, _BATCH), jnp.float32),
)


def kernel(users, items, user_table, item_table, W1, b1, W2, b2):
    uidx = users.reshape(_NW * _NK, _CHUNK)
    iidx = items.reshape(_NW * _NK, _CHUNK)
    ustags = [s.reshape(4, _RPAD) for s in _detile(user_table.T)]
    u_t = _sc_gather(*ustags, uidx)
    istags = [s.reshape(4, _RPAD) for s in _detile(item_table.T)]
    i_t = _sc_gather(*istags, iidx)
    w1u = W1[:, :_EMB]
    w1i = W1[:, _EMB:]
    out = _mlp(w1u, w1i, b1.reshape(_HID, 1), W2.reshape(1, _HID),
               b2.reshape(1, 1), u_t, i_t)
    return out.reshape(_BATCH)
